# Initial kernel scaffold; baseline (speedup 1.0000x reference)
import jax
import jax.numpy as jnp
from jax.experimental import pallas as pl


def kernel(t, gamma):
    raise NotImplementedError("write your pallas kernel here")


# speedup vs baseline: 4.5226x; 4.5226x over previous
import functools

import jax
import jax.numpy as jnp
from jax import lax
from jax.experimental import pallas as pl
from jax.experimental.pallas import tpu as pltpu
from jax.experimental.pallas import tpu_sc as plsc

N = 16384
TABLE = 1001
LANES = 16
NUM_CORES = 2
NUM_SUBCORES = 16
NUM_WORKERS = NUM_CORES * NUM_SUBCORES
CHUNK = N // NUM_WORKERS

_MAGIC = 8388608.0

_mesh = plsc.VectorSubcoreMesh(core_axis_name="c", subcore_axis_name="s")


@functools.partial(
    pl.kernel,
    mesh=_mesh,
    out_type=jax.ShapeDtypeStruct((N,), jnp.float32),
    scratch_types=[
        pltpu.VMEM((TABLE,), jnp.float32),
        pltpu.VMEM((CHUNK,), jnp.float32),
        pltpu.VMEM((CHUNK,), jnp.float32),
    ],
    compiler_params=pltpu.CompilerParams(needs_layout_passes=False),
)
def _gamma_lookup(t_hbm, gamma_hbm, out_hbm, tab_v, t_v, o_v):
    wid = lax.axis_index("s") * NUM_CORES + lax.axis_index("c")
    base = wid * CHUNK
    pltpu.sync_copy(gamma_hbm, tab_v)
    pltpu.sync_copy(t_hbm.at[pl.ds(base, CHUNK)], t_v)
    for j in range(CHUNK // LANES):
        tv = t_v[pl.ds(j * LANES, LANES)]
        idx = ((tv * 1000.0 + _MAGIC) - _MAGIC).astype(jnp.int32)
        o_v[pl.ds(j * LANES, LANES)] = plsc.load_gather(tab_v, [idx])
    pltpu.sync_copy(o_v, out_hbm.at[pl.ds(base, CHUNK)])


def kernel(t, gamma):
    out = _gamma_lookup(t.reshape(N), gamma)
    return out.reshape(N, 1)


# speedup vs baseline: 4.6141x; 1.0202x over previous
import functools

import jax
import jax.numpy as jnp
from jax import lax
from jax.experimental import pallas as pl
from jax.experimental.pallas import tpu as pltpu
from jax.experimental.pallas import tpu_sc as plsc

N = 16384
TABLE = 1001
LANES = 16
NUM_CORES = 2
NUM_SUBCORES = 16
NUM_WORKERS = NUM_CORES * NUM_SUBCORES
CHUNK = N // NUM_WORKERS

_MAGIC = 8388608.0

_mesh = plsc.VectorSubcoreMesh(core_axis_name="c", subcore_axis_name="s")


@functools.partial(
    pl.kernel,
    mesh=_mesh,
    out_type=jax.ShapeDtypeStruct((N,), jnp.float32),
    scratch_types=[
        pltpu.VMEM((TABLE,), jnp.float32),
        pltpu.VMEM((CHUNK,), jnp.float32),
        pltpu.VMEM((CHUNK,), jnp.float32),
        pltpu.SemaphoreType.DMA,
        pltpu.SemaphoreType.DMA,
    ],
    compiler_params=pltpu.CompilerParams(needs_layout_passes=False),
)
def _gamma_lookup(t_hbm, gamma_hbm, out_hbm, tab_v, t_v, o_v, sem_a, sem_b):
    wid = lax.axis_index("s") * NUM_CORES + lax.axis_index("c")
    base = wid * CHUNK
    cp_tab = pltpu.async_copy(gamma_hbm, tab_v, sem_a)
    cp_t = pltpu.async_copy(t_hbm.at[pl.ds(base, CHUNK)], t_v, sem_b)
    cp_tab.wait()
    cp_t.wait()
    for j in range(CHUNK // LANES):
        tv = t_v[pl.ds(j * LANES, LANES)]
        idx = ((tv * 1000.0 + _MAGIC) - _MAGIC).astype(jnp.int32)
        o_v[pl.ds(j * LANES, LANES)] = plsc.load_gather(tab_v, [idx])
    pltpu.sync_copy(o_v, out_hbm.at[pl.ds(base, CHUNK)])


def kernel(t, gamma):
    out = _gamma_lookup(t.reshape(N), gamma)
    return out.reshape(N, 1)


# speedup vs baseline: 5.0280x; 1.0897x over previous
import functools

import jax
import jax.numpy as jnp
from jax import lax
from jax.experimental import pallas as pl
from jax.experimental.pallas import tpu as pltpu
from jax.experimental.pallas import tpu_sc as plsc

N = 16384
TABLE = 1001
LANES = 16
NUM_CORES = 1
NUM_SUBCORES = 16
NUM_WORKERS = NUM_CORES * NUM_SUBCORES
CHUNK = N // NUM_WORKERS

_MAGIC = 8388608.0

_mesh = plsc.VectorSubcoreMesh(
    core_axis_name="c", subcore_axis_name="s", num_cores=NUM_CORES
)


@functools.partial(
    pl.kernel,
    mesh=_mesh,
    out_type=jax.ShapeDtypeStruct((N,), jnp.float32),
    scratch_types=[
        pltpu.VMEM((TABLE,), jnp.float32),
        pltpu.VMEM((CHUNK,), jnp.float32),
        pltpu.VMEM((CHUNK,), jnp.float32),
        pltpu.SemaphoreType.DMA,
        pltpu.SemaphoreType.DMA,
    ],
    compiler_params=pltpu.CompilerParams(needs_layout_passes=False),
)
def _gamma_lookup(t_hbm, gamma_hbm, out_hbm, tab_v, t_v, o_v, sem_a, sem_b):
    wid = lax.axis_index("s") * NUM_CORES + lax.axis_index("c")
    base = wid * CHUNK
    cp_tab = pltpu.async_copy(gamma_hbm, tab_v, sem_a)
    cp_t = pltpu.async_copy(t_hbm.at[pl.ds(base, CHUNK)], t_v, sem_b)
    cp_tab.wait()
    cp_t.wait()
    for j in range(CHUNK // LANES):
        tv = t_v[pl.ds(j * LANES, LANES)]
        idx = ((tv * 1000.0 + _MAGIC) - _MAGIC).astype(jnp.int32)
        o_v[pl.ds(j * LANES, LANES)] = plsc.load_gather(tab_v, [idx])
    pltpu.sync_copy(o_v, out_hbm.at[pl.ds(base, CHUNK)])


def kernel(t, gamma):
    out = _gamma_lookup(t.reshape(N), gamma)
    return out.reshape(N, 1)
